# baseline (speedup 1.0000x reference)
import functools

import jax
import jax.numpy as jnp
from jax import lax
from jax.experimental import pallas as pl
from jax.experimental.pallas import tpu as pltpu
from jax.experimental.pallas import tpu_sc as plsc

_NUM_BANDS = 64
_EMBED_DIM = 128
_B = 4096
_S = 2048
_NC = 2
_NS = 16
_NW = _NC * _NS
_BPW = _S // _NW
_BLOCK_B = 256

_mesh = plsc.VectorSubcoreMesh(core_axis_name="c", subcore_axis_name="s")


@functools.partial(
    pl.kernel,
    mesh=_mesh,
    out_type=jax.ShapeDtypeStruct((_B, _NUM_BANDS, _EMBED_DIM), jnp.float32),
    scratch_types=[
        pltpu.VMEM((_NUM_BANDS,), jnp.int32),
        pltpu.VMEM((1, _NUM_BANDS, _EMBED_DIM), jnp.float32),
        pltpu.SemaphoreType.DMA,
        pltpu.SemaphoreType.DMA,
    ],
)
def _sc_stage(table_hbm, out_hbm, idx_v, buf, gsem, sem):
    wid = lax.axis_index("s") * _NC + lax.axis_index("c")
    base = wid * _BPW
    for j in range(_NUM_BANDS // 16):
        idx_v[pl.ds(16 * j, 16)] = lax.iota(jnp.int32, 16) + 16 * j
    pltpu.async_copy(table_hbm.at[idx_v], buf.at[0], gsem).wait()
    copies = []
    for i in range(_BPW):
        copies.append(
            pltpu.async_copy(buf, out_hbm.at[pl.ds(base + i, 1)], sem)
        )
    for c in copies:
        c.wait()


def _tc_body(table_ref, part_ref, out_ref):
    del part_ref
    out_ref[...] = jnp.broadcast_to(
        table_ref[...][None], (_BLOCK_B, _NUM_BANDS, _EMBED_DIM)
    )


@jax.jit
def _fill_rest_tc(table, part):
    return pl.pallas_call(
        _tc_body,
        grid=((_B - _S) // _BLOCK_B,),
        in_specs=[
            pl.BlockSpec((_NUM_BANDS, _EMBED_DIM), lambda i: (0, 0)),
            pl.BlockSpec(memory_space=pl.ANY),
        ],
        out_specs=pl.BlockSpec(
            (_BLOCK_B, _NUM_BANDS, _EMBED_DIM),
            lambda i: (i + _S // _BLOCK_B, 0, 0),
        ),
        out_shape=jax.ShapeDtypeStruct((_B, _NUM_BANDS, _EMBED_DIM), jnp.float32),
        input_output_aliases={1: 0},
    )(table, part)


def kernel(embedding_weight, batch_size):
    del batch_size
    part = _sc_stage(embedding_weight)
    return _fill_rest_tc(embedding_weight, part)


# speedup vs baseline: 1.0809x; 1.0809x over previous
import functools

import jax
import jax.numpy as jnp
from jax import lax
from jax.experimental import pallas as pl
from jax.experimental.pallas import tpu as pltpu
from jax.experimental.pallas import tpu_sc as plsc

_NUM_BANDS = 64
_EMBED_DIM = 128
_B = 4096
_BLOCK_B = 256

_smesh = plsc.ScalarSubcoreMesh(axis_name="c", num_cores=2)


@functools.partial(
    pl.kernel,
    mesh=_smesh,
    out_type=jax.ShapeDtypeStruct((_NUM_BANDS, _EMBED_DIM), jnp.float32),
    scratch_types=[
        pltpu.VMEM_SHARED((_NUM_BANDS, _EMBED_DIM), jnp.float32),
    ],
)
def _lookup_scs(table_hbm, out_hbm, stage):
    cid = lax.axis_index("c")

    @pl.when(cid == 0)
    def _():
        pltpu.sync_copy(table_hbm, stage)
        pltpu.sync_copy(stage, out_hbm)


def _tc_body(table_ref, out_ref):
    out_ref[...] = jnp.broadcast_to(
        table_ref[...][None], (_BLOCK_B, _NUM_BANDS, _EMBED_DIM)
    )


@jax.jit
def _broadcast_tc(table):
    return pl.pallas_call(
        _tc_body,
        grid=(_B // _BLOCK_B,),
        in_specs=[
            pl.BlockSpec((_NUM_BANDS, _EMBED_DIM), lambda i: (0, 0)),
        ],
        out_specs=pl.BlockSpec(
            (_BLOCK_B, _NUM_BANDS, _EMBED_DIM), lambda i: (i, 0, 0)
        ),
        out_shape=jax.ShapeDtypeStruct((_B, _NUM_BANDS, _EMBED_DIM), jnp.float32),
    )(table)


def kernel(embedding_weight, batch_size):
    del batch_size
    return _broadcast_tc(_lookup_scs(embedding_weight))
